# baseline (speedup 1.0000x reference)
import functools

import jax
import jax.numpy as jnp
from jax import lax
from jax.experimental import pallas as pl
from jax.experimental.pallas import tpu as pltpu
from jax.experimental.pallas import tpu_sc as plsc

N_NODES = 10000
IN_CH = 128
MID_CH = 64
OUT_CH = 128

NC = 2
NS = 16
NW = NC * NS
CHUNK = 128
CHUNKS = 80
HALF_CHUNKS = 40
TOTAL_CHUNKS = NW * CHUNKS
SEG_CH = (80, 80)
E_PAD = NW * CHUNKS * CHUNK
N_PAD = 10240
N_SC = 10112
ROWS_PER_TILE = N_SC // NS
SCRAP = N_NODES

_mesh = plsc.VectorSubcoreMesh(core_axis_name="c", subcore_axis_name="s")

_ROW_BLOCKS = [(0, 128), (128, 128), (256, 128), (384, 128), (512, 120)]


DEG_W = 128


@functools.partial(
    pl.kernel,
    mesh=_mesh,
    out_type=jax.ShapeDtypeStruct((NC, N_PAD, DEG_W), jnp.float32),
    scratch_types=[
        pltpu.VMEM((CHUNKS, CHUNK), jnp.int32),
        pltpu.VMEM((CHUNK, DEG_W), jnp.float32),
        pltpu.VMEM((CHUNK, DEG_W), jnp.float32),
        pltpu.VMEM_SHARED((N_SC, DEG_W), jnp.float32),
        pltpu.SemaphoreType.DMA,
    ],
)
def _deg_kernel(dsts_hbm, ones_hbm, zeros_hbm, out_hbm, dst_v, ones_v, zbuf, S_sh,
                sem_s):
    cid = lax.axis_index("c")
    sid = lax.axis_index("s")
    wid = sid * NC + cid

    pltpu.sync_copy(zeros_hbm, zbuf)
    pltpu.sync_copy(ones_hbm, ones_v)
    pltpu.sync_copy(dsts_hbm.at[wid], dst_v)
    base = sid * ROWS_PER_TILE
    for off, sz in _ROW_BLOCKS:
        pltpu.sync_copy(zbuf.at[pl.ds(0, sz)], S_sh.at[pl.ds(base + off, sz)])
    plsc.subcore_barrier()

    def body(j, carry):
        pltpu.async_copy(ones_v, S_sh.at[dst_v.at[j]], sem_s, add=True)
        return carry

    lax.fori_loop(0, CHUNKS, body, 0)

    def drain(j, carry):
        pltpu.make_async_copy(ones_v, S_sh.at[dst_v.at[j]], sem_s).wait()
        return carry

    lax.fori_loop(0, CHUNKS, drain, 0)
    plsc.subcore_barrier()
    for off, sz in _ROW_BLOCKS:
        pltpu.sync_copy(S_sh.at[pl.ds(base + off, sz)], zbuf.at[pl.ds(0, sz)])
        pltpu.sync_copy(zbuf.at[pl.ds(0, sz)], out_hbm.at[cid, pl.ds(base + off, sz)])


def _pieces(n):
    return [min(HALF_CHUNKS, n - i) for i in range(0, n, HALF_CHUNKS)]


def _make_seg_kernel(C):
    @functools.partial(
        pl.kernel,
        mesh=_mesh,
        out_type=jax.ShapeDtypeStruct((NC, N_PAD, C), jnp.float32),
        scratch_types=[
            pltpu.VMEM((HALF_CHUNKS, CHUNK), jnp.int32),
            pltpu.VMEM((HALF_CHUNKS, CHUNK), jnp.int32),
            pltpu.VMEM((CHUNK, C), jnp.float32),
            pltpu.VMEM((CHUNK, C), jnp.float32),
            pltpu.VMEM_SHARED((N_SC, C), jnp.float32),
            pltpu.SemaphoreType.DMA,
            pltpu.SemaphoreType.DMA,
            pltpu.SemaphoreType.DMA,
            pltpu.SemaphoreType.DMA,
        ],
    )
    def _seg(xl0_hbm, xl1_hbm, srcs_hbm, dsts_hbm, zeros_hbm, out_hbm,
             src_v, dst_v, buf_a, buf_b, S_sh, sem_ga, sem_gb, sem_sa, sem_sb):
        cid = lax.axis_index("c")
        sid = lax.axis_index("s")
        base = sid * ROWS_PER_TILE

        pltpu.sync_copy(zeros_hbm, buf_a)
        for off, sz in _ROW_BLOCKS:
            pltpu.sync_copy(buf_a.at[pl.ds(0, sz)], S_sh.at[pl.ds(base + off, sz)])
        plsc.subcore_barrier()

        def run_piece(xl_hbm, chunk0, pn):
            pltpu.sync_copy(srcs_hbm.at[pl.ds(chunk0, pn)], src_v.at[pl.ds(0, pn)])
            pltpu.sync_copy(dsts_hbm.at[pl.ds(chunk0, pn)], dst_v.at[pl.ds(0, pn)])
            pltpu.async_copy(xl_hbm.at[src_v.at[0]], buf_a, sem_ga)

            def body(jj, carry):
                j = 2 * jj

                @pl.when(jj > 0)
                def _():
                    pltpu.make_async_copy(buf_b, S_sh.at[dst_v.at[j - 1]],
                                          sem_sb).wait()

                pltpu.async_copy(xl_hbm.at[src_v.at[j + 1]], buf_b, sem_gb)
                pltpu.make_async_copy(xl_hbm.at[src_v.at[j]], buf_a, sem_ga).wait()
                pltpu.async_copy(buf_a, S_sh.at[dst_v.at[j]], sem_sa, add=True)
                pltpu.make_async_copy(buf_a, S_sh.at[dst_v.at[j]], sem_sa).wait()

                @pl.when(jj < pn // 2 - 1)
                def _():
                    pltpu.async_copy(xl_hbm.at[src_v.at[j + 2]], buf_a, sem_ga)

                pltpu.make_async_copy(xl_hbm.at[src_v.at[j + 1]], buf_b, sem_gb).wait()
                pltpu.async_copy(buf_b, S_sh.at[dst_v.at[j + 1]], sem_sb, add=True)
                return carry

            lax.fori_loop(0, pn // 2, body, 0)
            pltpu.make_async_copy(buf_b, S_sh.at[dst_v.at[pn - 1]], sem_sb).wait()

        for c in range(NC):
            ch = SEG_CH[c]
            core_base = 0 if c == 0 else NS * SEG_CH[0]

            @pl.when(cid == c)
            def _(ch=ch, core_base=core_base):
                tile0 = core_base + sid * ch
                off = 0
                xl_hbm = xl0_hbm if c == 0 else xl1_hbm
                for pn in _pieces(ch):
                    run_piece(xl_hbm, tile0 + off, pn)
                    off += pn

        plsc.subcore_barrier()
        for off, sz in _ROW_BLOCKS:
            pltpu.sync_copy(S_sh.at[pl.ds(base + off, sz)], buf_a.at[pl.ds(0, sz)])
            pltpu.sync_copy(buf_a.at[pl.ds(0, sz)], out_hbm.at[cid, pl.ds(base + off, sz)])

    return _seg


_seg128 = _make_seg_kernel(OUT_CH)


TC_R = 1024


def _tc1_body(x_ref, w_ref, degp_ref, xl_ref, xlb_ref, dis_ref):
    deg = degp_ref[0, :, 0:1] + degp_ref[1, :, 0:1] + 1.0
    dis = lax.rsqrt(deg)
    xl = jnp.dot(x_ref[...], w_ref[...], preferred_element_type=jnp.float32)
    xlp = jnp.concatenate(
        [xl * dis, jnp.zeros((TC_R, OUT_CH - MID_CH), jnp.float32)], axis=1)
    xl_ref[...] = xlp
    xlb_ref[...] = xlp
    dis_ref[...] = dis


def _tc1(x, W1, degp):
    grid = N_PAD // TC_R
    return pl.pallas_call(
        _tc1_body,
        grid=(grid,),
        in_specs=[
            pl.BlockSpec((TC_R, IN_CH), lambda i: (i, 0)),
            pl.BlockSpec((IN_CH, MID_CH), lambda i: (0, 0)),
            pl.BlockSpec((NC, TC_R, DEG_W), lambda i: (0, i, 0)),
        ],
        out_specs=[
            pl.BlockSpec((TC_R, OUT_CH), lambda i: (i, 0)),
            pl.BlockSpec((TC_R, OUT_CH), lambda i: (i, 0)),
            pl.BlockSpec((TC_R, 1), lambda i: (i, 0)),
        ],
        out_shape=[
            jax.ShapeDtypeStruct((N_PAD, OUT_CH), jnp.float32),
            jax.ShapeDtypeStruct((N_PAD, OUT_CH), jnp.float32),
            jax.ShapeDtypeStruct((N_PAD, 1), jnp.float32),
        ],
    )(x, W1, degp)


def _tc2_body(s_ref, xl_ref, dis_ref, b_ref, w_ref, out_ref, outb_ref):
    dis = dis_ref[...]
    t = (s_ref[0] + s_ref[1] + xl_ref[...])[:, :MID_CH]
    z = dis * t + b_ref[...]
    h = jnp.where(z > 0, z, jnp.exp(z) - 1.0)
    o = jnp.dot(h, w_ref[...], preferred_element_type=jnp.float32) * dis
    out_ref[...] = o
    outb_ref[...] = o


def _tc2(S1, xl1, dis, b1, W2):
    grid = N_PAD // TC_R
    return pl.pallas_call(
        _tc2_body,
        grid=(grid,),
        in_specs=[
            pl.BlockSpec((NC, TC_R, OUT_CH), lambda i: (0, i, 0)),
            pl.BlockSpec((TC_R, OUT_CH), lambda i: (i, 0)),
            pl.BlockSpec((TC_R, 1), lambda i: (i, 0)),
            pl.BlockSpec((1, MID_CH), lambda i: (0, 0)),
            pl.BlockSpec((MID_CH, OUT_CH), lambda i: (0, 0)),
        ],
        out_specs=[pl.BlockSpec((TC_R, OUT_CH), lambda i: (i, 0)),
                   pl.BlockSpec((TC_R, OUT_CH), lambda i: (i, 0))],
        out_shape=[jax.ShapeDtypeStruct((N_PAD, OUT_CH), jnp.float32),
                   jax.ShapeDtypeStruct((N_PAD, OUT_CH), jnp.float32)],
    )(S1, xl1, dis, b1, W2)


def _tc3_body(s_ref, xl_ref, dis_ref, b_ref, out_ref):
    z = dis_ref[...] * (s_ref[0] + s_ref[1] + xl_ref[...]) + b_ref[...]
    m = jnp.max(z, axis=1, keepdims=True)
    e = jnp.exp(z - m)
    out_ref[...] = (z - m) - jnp.log(jnp.sum(e, axis=1, keepdims=True))


def _tc3(S2, xl2, dis, b2):
    grid = N_PAD // TC_R
    return pl.pallas_call(
        _tc3_body,
        grid=(grid,),
        in_specs=[
            pl.BlockSpec((NC, TC_R, OUT_CH), lambda i: (0, i, 0)),
            pl.BlockSpec((TC_R, OUT_CH), lambda i: (i, 0)),
            pl.BlockSpec((TC_R, 1), lambda i: (i, 0)),
            pl.BlockSpec((1, OUT_CH), lambda i: (0, 0)),
        ],
        out_specs=pl.BlockSpec((TC_R, OUT_CH), lambda i: (i, 0)),
        out_shape=jax.ShapeDtypeStruct((N_PAD, OUT_CH), jnp.float32),
    )(S2, xl2, dis, b2)


def kernel(node_feature, adj_list, W1, b1, W2, b2):
    src = adj_list[0].astype(jnp.int32)
    dst = adj_list[1].astype(jnp.int32)
    n_edges = src.shape[0]
    pad = E_PAD - n_edges
    src = jnp.concatenate([src, jnp.full((pad,), SCRAP, jnp.int32)])
    dst = jnp.concatenate([dst, jnp.full((pad,), SCRAP, jnp.int32)])
    srcs3 = src.reshape(NW, CHUNKS, CHUNK)
    dsts3 = dst.reshape(NW, CHUNKS, CHUNK)
    srcs2 = src.reshape(TOTAL_CHUNKS, CHUNK)
    dsts2 = dst.reshape(TOTAL_CHUNKS, CHUNK)

    x = jnp.zeros((N_PAD, IN_CH), jnp.float32).at[:N_NODES].set(node_feature)
    ones128 = jnp.ones((CHUNK, DEG_W), jnp.float32)
    zeros128 = jnp.zeros((CHUNK, OUT_CH), jnp.float32)

    degp = _deg_kernel(dsts3, ones128, zeros128)
    xl1, xl1b, dis = _tc1(x, W1, degp)
    S1 = _seg128(xl1, xl1b, srcs2, dsts2, zeros128)
    xl2, xl2b = _tc2(S1, xl1, dis, b1.reshape(1, MID_CH), W2)
    S2 = _seg128(xl2, xl2b, srcs2, dsts2, zeros128)
    out = _tc3(S2, xl2, dis, b2.reshape(1, OUT_CH))
    return out[:N_NODES]


# speedup vs baseline: 1.0257x; 1.0257x over previous
import functools

import jax
import jax.numpy as jnp
from jax import lax
from jax.experimental import pallas as pl
from jax.experimental.pallas import tpu as pltpu
from jax.experimental.pallas import tpu_sc as plsc

N_NODES = 10000
IN_CH = 128
MID_CH = 64
OUT_CH = 128

NC = 2
NS = 16
NW = NC * NS
CHUNK = 128
CHUNKS = 80
HALF_CHUNKS = 40
TOTAL_CHUNKS = NW * CHUNKS
SEG_CH = (112, 48)
E_PAD = NW * CHUNKS * CHUNK
N_PAD = 10240
N_SC = 10112
ROWS_PER_TILE = N_SC // NS
SCRAP = N_NODES

_mesh = plsc.VectorSubcoreMesh(core_axis_name="c", subcore_axis_name="s")

_ROW_BLOCKS = [(0, 128), (128, 128), (256, 128), (384, 128), (512, 120)]


DEG_W = 128


@functools.partial(
    pl.kernel,
    mesh=_mesh,
    out_type=jax.ShapeDtypeStruct((NC, N_PAD, DEG_W), jnp.float32),
    scratch_types=[
        pltpu.VMEM((CHUNKS, CHUNK), jnp.int32),
        pltpu.VMEM((CHUNK, DEG_W), jnp.float32),
        pltpu.VMEM((CHUNK, DEG_W), jnp.float32),
        pltpu.VMEM_SHARED((N_SC, DEG_W), jnp.float32),
        pltpu.SemaphoreType.DMA,
    ],
)
def _deg_kernel(dsts_hbm, ones_hbm, zeros_hbm, out_hbm, dst_v, ones_v, zbuf, S_sh,
                sem_s):
    cid = lax.axis_index("c")
    sid = lax.axis_index("s")
    wid = sid * NC + cid

    pltpu.sync_copy(zeros_hbm, zbuf)
    pltpu.sync_copy(ones_hbm, ones_v)
    pltpu.sync_copy(dsts_hbm.at[wid], dst_v)
    base = sid * ROWS_PER_TILE
    for off, sz in _ROW_BLOCKS:
        pltpu.sync_copy(zbuf.at[pl.ds(0, sz)], S_sh.at[pl.ds(base + off, sz)])
    plsc.subcore_barrier()

    def body(j, carry):
        pltpu.async_copy(ones_v, S_sh.at[dst_v.at[j]], sem_s, add=True)
        return carry

    lax.fori_loop(0, CHUNKS, body, 0)

    def drain(j, carry):
        pltpu.make_async_copy(ones_v, S_sh.at[dst_v.at[j]], sem_s).wait()
        return carry

    lax.fori_loop(0, CHUNKS, drain, 0)
    plsc.subcore_barrier()
    for off, sz in _ROW_BLOCKS:
        pltpu.sync_copy(S_sh.at[pl.ds(base + off, sz)], zbuf.at[pl.ds(0, sz)])
        pltpu.sync_copy(zbuf.at[pl.ds(0, sz)], out_hbm.at[cid, pl.ds(base + off, sz)])


def _pieces(n):
    return [min(HALF_CHUNKS, n - i) for i in range(0, n, HALF_CHUNKS)]


def _make_seg_kernel(C):
    @functools.partial(
        pl.kernel,
        mesh=_mesh,
        out_type=jax.ShapeDtypeStruct((NC, N_PAD, C), jnp.float32),
        scratch_types=[
            pltpu.VMEM((HALF_CHUNKS, CHUNK), jnp.int32),
            pltpu.VMEM((HALF_CHUNKS, CHUNK), jnp.int32),
            pltpu.VMEM((CHUNK, C), jnp.float32),
            pltpu.VMEM((CHUNK, C), jnp.float32),
            pltpu.VMEM_SHARED((N_SC, C), jnp.float32),
            pltpu.SemaphoreType.DMA,
            pltpu.SemaphoreType.DMA,
            pltpu.SemaphoreType.DMA,
            pltpu.SemaphoreType.DMA,
        ],
    )
    def _seg(xl0_hbm, xl1_hbm, srcs_hbm, dsts_hbm, zeros_hbm, out_hbm,
             src_v, dst_v, buf_a, buf_b, S_sh, sem_ga, sem_gb, sem_sa, sem_sb):
        cid = lax.axis_index("c")
        sid = lax.axis_index("s")
        base = sid * ROWS_PER_TILE

        pltpu.sync_copy(zeros_hbm, buf_a)
        for off, sz in _ROW_BLOCKS:
            pltpu.sync_copy(buf_a.at[pl.ds(0, sz)], S_sh.at[pl.ds(base + off, sz)])
        plsc.subcore_barrier()

        def run_piece(xl_hbm, chunk0, pn):
            pltpu.sync_copy(srcs_hbm.at[pl.ds(chunk0, pn)], src_v.at[pl.ds(0, pn)])
            pltpu.sync_copy(dsts_hbm.at[pl.ds(chunk0, pn)], dst_v.at[pl.ds(0, pn)])
            pltpu.async_copy(xl_hbm.at[src_v.at[0]], buf_a, sem_ga)

            def body(jj, carry):
                j = 2 * jj

                @pl.when(jj > 0)
                def _():
                    pltpu.make_async_copy(buf_b, S_sh.at[dst_v.at[j - 1]],
                                          sem_sb).wait()

                pltpu.async_copy(xl_hbm.at[src_v.at[j + 1]], buf_b, sem_gb)
                pltpu.make_async_copy(xl_hbm.at[src_v.at[j]], buf_a, sem_ga).wait()
                pltpu.async_copy(buf_a, S_sh.at[dst_v.at[j]], sem_sa, add=True)
                pltpu.make_async_copy(buf_a, S_sh.at[dst_v.at[j]], sem_sa).wait()

                @pl.when(jj < pn // 2 - 1)
                def _():
                    pltpu.async_copy(xl_hbm.at[src_v.at[j + 2]], buf_a, sem_ga)

                pltpu.make_async_copy(xl_hbm.at[src_v.at[j + 1]], buf_b, sem_gb).wait()
                pltpu.async_copy(buf_b, S_sh.at[dst_v.at[j + 1]], sem_sb, add=True)
                return carry

            lax.fori_loop(0, pn // 2, body, 0)
            pltpu.make_async_copy(buf_b, S_sh.at[dst_v.at[pn - 1]], sem_sb).wait()

        for c in range(NC):
            ch = SEG_CH[c]
            core_base = 0 if c == 0 else NS * SEG_CH[0]

            @pl.when(cid == c)
            def _(ch=ch, core_base=core_base):
                tile0 = core_base + sid * ch
                off = 0
                xl_hbm = xl0_hbm if c == 0 else xl1_hbm
                for pn in _pieces(ch):
                    run_piece(xl_hbm, tile0 + off, pn)
                    off += pn

        plsc.subcore_barrier()
        for off, sz in _ROW_BLOCKS:
            pltpu.sync_copy(S_sh.at[pl.ds(base + off, sz)], buf_a.at[pl.ds(0, sz)])
            pltpu.sync_copy(buf_a.at[pl.ds(0, sz)], out_hbm.at[cid, pl.ds(base + off, sz)])

    return _seg


_seg128 = _make_seg_kernel(OUT_CH)


TC_R = 1024


def _tc1_body(x_ref, w_ref, degp_ref, xl_ref, xlb_ref, dis_ref):
    deg = degp_ref[0, :, 0:1] + degp_ref[1, :, 0:1] + 1.0
    dis = lax.rsqrt(deg)
    xl = jnp.dot(x_ref[...], w_ref[...], preferred_element_type=jnp.float32)
    xlp = jnp.concatenate(
        [xl * dis, jnp.zeros((TC_R, OUT_CH - MID_CH), jnp.float32)], axis=1)
    xl_ref[...] = xlp
    xlb_ref[...] = xlp
    dis_ref[...] = dis


def _tc1(x, W1, degp):
    grid = N_PAD // TC_R
    return pl.pallas_call(
        _tc1_body,
        grid=(grid,),
        in_specs=[
            pl.BlockSpec((TC_R, IN_CH), lambda i: (i, 0)),
            pl.BlockSpec((IN_CH, MID_CH), lambda i: (0, 0)),
            pl.BlockSpec((NC, TC_R, DEG_W), lambda i: (0, i, 0)),
        ],
        out_specs=[
            pl.BlockSpec((TC_R, OUT_CH), lambda i: (i, 0)),
            pl.BlockSpec((TC_R, OUT_CH), lambda i: (i, 0)),
            pl.BlockSpec((TC_R, 1), lambda i: (i, 0)),
        ],
        out_shape=[
            jax.ShapeDtypeStruct((N_PAD, OUT_CH), jnp.float32),
            jax.ShapeDtypeStruct((N_PAD, OUT_CH), jnp.float32),
            jax.ShapeDtypeStruct((N_PAD, 1), jnp.float32),
        ],
    )(x, W1, degp)


def _tc2_body(s_ref, xl_ref, dis_ref, b_ref, w_ref, out_ref, outb_ref):
    dis = dis_ref[...]
    t = (s_ref[0] + s_ref[1] + xl_ref[...])[:, :MID_CH]
    z = dis * t + b_ref[...]
    h = jnp.where(z > 0, z, jnp.exp(z) - 1.0)
    o = jnp.dot(h, w_ref[...], preferred_element_type=jnp.float32) * dis
    out_ref[...] = o
    outb_ref[...] = o


def _tc2(S1, xl1, dis, b1, W2):
    grid = N_PAD // TC_R
    return pl.pallas_call(
        _tc2_body,
        grid=(grid,),
        in_specs=[
            pl.BlockSpec((NC, TC_R, OUT_CH), lambda i: (0, i, 0)),
            pl.BlockSpec((TC_R, OUT_CH), lambda i: (i, 0)),
            pl.BlockSpec((TC_R, 1), lambda i: (i, 0)),
            pl.BlockSpec((1, MID_CH), lambda i: (0, 0)),
            pl.BlockSpec((MID_CH, OUT_CH), lambda i: (0, 0)),
        ],
        out_specs=[pl.BlockSpec((TC_R, OUT_CH), lambda i: (i, 0)),
                   pl.BlockSpec((TC_R, OUT_CH), lambda i: (i, 0))],
        out_shape=[jax.ShapeDtypeStruct((N_PAD, OUT_CH), jnp.float32),
                   jax.ShapeDtypeStruct((N_PAD, OUT_CH), jnp.float32)],
    )(S1, xl1, dis, b1, W2)


def _tc3_body(s_ref, xl_ref, dis_ref, b_ref, out_ref):
    z = dis_ref[...] * (s_ref[0] + s_ref[1] + xl_ref[...]) + b_ref[...]
    m = jnp.max(z, axis=1, keepdims=True)
    e = jnp.exp(z - m)
    out_ref[...] = (z - m) - jnp.log(jnp.sum(e, axis=1, keepdims=True))


def _tc3(S2, xl2, dis, b2):
    grid = N_PAD // TC_R
    return pl.pallas_call(
        _tc3_body,
        grid=(grid,),
        in_specs=[
            pl.BlockSpec((NC, TC_R, OUT_CH), lambda i: (0, i, 0)),
            pl.BlockSpec((TC_R, OUT_CH), lambda i: (i, 0)),
            pl.BlockSpec((TC_R, 1), lambda i: (i, 0)),
            pl.BlockSpec((1, OUT_CH), lambda i: (0, 0)),
        ],
        out_specs=pl.BlockSpec((TC_R, OUT_CH), lambda i: (i, 0)),
        out_shape=jax.ShapeDtypeStruct((N_PAD, OUT_CH), jnp.float32),
    )(S2, xl2, dis, b2)


def kernel(node_feature, adj_list, W1, b1, W2, b2):
    src = adj_list[0].astype(jnp.int32)
    dst = adj_list[1].astype(jnp.int32)
    n_edges = src.shape[0]
    pad = E_PAD - n_edges
    src = jnp.concatenate([src, jnp.full((pad,), SCRAP, jnp.int32)])
    dst = jnp.concatenate([dst, jnp.full((pad,), SCRAP, jnp.int32)])
    srcs3 = src.reshape(NW, CHUNKS, CHUNK)
    dsts3 = dst.reshape(NW, CHUNKS, CHUNK)
    srcs2 = src.reshape(TOTAL_CHUNKS, CHUNK)
    dsts2 = dst.reshape(TOTAL_CHUNKS, CHUNK)

    x = jnp.zeros((N_PAD, IN_CH), jnp.float32).at[:N_NODES].set(node_feature)
    ones128 = jnp.ones((CHUNK, DEG_W), jnp.float32)
    zeros128 = jnp.zeros((CHUNK, OUT_CH), jnp.float32)

    degp = _deg_kernel(dsts3, ones128, zeros128)
    xl1, xl1b, dis = _tc1(x, W1, degp)
    S1 = _seg128(xl1, xl1b, srcs2, dsts2, zeros128)
    xl2, xl2b = _tc2(S1, xl1, dis, b1.reshape(1, MID_CH), W2)
    S2 = _seg128(xl2, xl2b, srcs2, dsts2, zeros128)
    out = _tc3(S2, xl2, dis, b2.reshape(1, OUT_CH))
    return out[:N_NODES]


# speedup vs baseline: 1.0366x; 1.0106x over previous
import functools

import jax
import jax.numpy as jnp
from jax import lax
from jax.experimental import pallas as pl
from jax.experimental.pallas import tpu as pltpu
from jax.experimental.pallas import tpu_sc as plsc

N_NODES = 10000
IN_CH = 128
MID_CH = 64
OUT_CH = 128

NC = 2
NS = 16
NW = NC * NS
CHUNK = 128
CHUNKS = 80
HALF_CHUNKS = 40
TOTAL_CHUNKS = NW * CHUNKS
SEG_CH = (136, 24)
E_PAD = NW * CHUNKS * CHUNK
N_PAD = 10240
N_SC = 10112
ROWS_PER_TILE = N_SC // NS
SCRAP = N_NODES

_mesh = plsc.VectorSubcoreMesh(core_axis_name="c", subcore_axis_name="s")

_ROW_BLOCKS = [(0, 128), (128, 128), (256, 128), (384, 128), (512, 120)]


DEG_W = 128


@functools.partial(
    pl.kernel,
    mesh=_mesh,
    out_type=jax.ShapeDtypeStruct((NC, N_PAD, DEG_W), jnp.float32),
    scratch_types=[
        pltpu.VMEM((CHUNKS, CHUNK), jnp.int32),
        pltpu.VMEM((CHUNK, DEG_W), jnp.float32),
        pltpu.VMEM((CHUNK, DEG_W), jnp.float32),
        pltpu.VMEM_SHARED((N_SC, DEG_W), jnp.float32),
        pltpu.SemaphoreType.DMA,
    ],
)
def _deg_kernel(dsts_hbm, ones_hbm, zeros_hbm, out_hbm, dst_v, ones_v, zbuf, S_sh,
                sem_s):
    cid = lax.axis_index("c")
    sid = lax.axis_index("s")
    wid = sid * NC + cid

    pltpu.sync_copy(zeros_hbm, zbuf)
    pltpu.sync_copy(ones_hbm, ones_v)
    pltpu.sync_copy(dsts_hbm.at[wid], dst_v)
    base = sid * ROWS_PER_TILE
    for off, sz in _ROW_BLOCKS:
        pltpu.sync_copy(zbuf.at[pl.ds(0, sz)], S_sh.at[pl.ds(base + off, sz)])
    plsc.subcore_barrier()

    def body(j, carry):
        pltpu.async_copy(ones_v, S_sh.at[dst_v.at[j]], sem_s, add=True)
        return carry

    lax.fori_loop(0, CHUNKS, body, 0)

    def drain(j, carry):
        pltpu.make_async_copy(ones_v, S_sh.at[dst_v.at[j]], sem_s).wait()
        return carry

    lax.fori_loop(0, CHUNKS, drain, 0)
    plsc.subcore_barrier()
    for off, sz in _ROW_BLOCKS:
        pltpu.sync_copy(S_sh.at[pl.ds(base + off, sz)], zbuf.at[pl.ds(0, sz)])
        pltpu.sync_copy(zbuf.at[pl.ds(0, sz)], out_hbm.at[cid, pl.ds(base + off, sz)])


def _pieces(n):
    return [min(HALF_CHUNKS, n - i) for i in range(0, n, HALF_CHUNKS)]


def _make_seg_kernel(C):
    @functools.partial(
        pl.kernel,
        mesh=_mesh,
        out_type=jax.ShapeDtypeStruct((NC, N_PAD, C), jnp.float32),
        scratch_types=[
            pltpu.VMEM((HALF_CHUNKS, CHUNK), jnp.int32),
            pltpu.VMEM((HALF_CHUNKS, CHUNK), jnp.int32),
            pltpu.VMEM((CHUNK, C), jnp.float32),
            pltpu.VMEM((CHUNK, C), jnp.float32),
            pltpu.VMEM_SHARED((N_SC, C), jnp.float32),
            pltpu.SemaphoreType.DMA,
            pltpu.SemaphoreType.DMA,
            pltpu.SemaphoreType.DMA,
            pltpu.SemaphoreType.DMA,
        ],
    )
    def _seg(xl0_hbm, xl1_hbm, srcs_hbm, dsts_hbm, zeros_hbm, out_hbm,
             src_v, dst_v, buf_a, buf_b, S_sh, sem_ga, sem_gb, sem_sa, sem_sb):
        cid = lax.axis_index("c")
        sid = lax.axis_index("s")
        base = sid * ROWS_PER_TILE

        pltpu.sync_copy(zeros_hbm, buf_a)
        for off, sz in _ROW_BLOCKS:
            pltpu.sync_copy(buf_a.at[pl.ds(0, sz)], S_sh.at[pl.ds(base + off, sz)])
        plsc.subcore_barrier()

        def run_piece(xl_hbm, chunk0, pn):
            pltpu.sync_copy(srcs_hbm.at[pl.ds(chunk0, pn)], src_v.at[pl.ds(0, pn)])
            pltpu.sync_copy(dsts_hbm.at[pl.ds(chunk0, pn)], dst_v.at[pl.ds(0, pn)])
            pltpu.async_copy(xl_hbm.at[src_v.at[0]], buf_a, sem_ga)

            def body(jj, carry):
                j = 2 * jj

                @pl.when(jj > 0)
                def _():
                    pltpu.make_async_copy(buf_b, S_sh.at[dst_v.at[j - 1]],
                                          sem_sb).wait()

                pltpu.async_copy(xl_hbm.at[src_v.at[j + 1]], buf_b, sem_gb)
                pltpu.make_async_copy(xl_hbm.at[src_v.at[j]], buf_a, sem_ga).wait()
                pltpu.async_copy(buf_a, S_sh.at[dst_v.at[j]], sem_sa, add=True)
                pltpu.make_async_copy(buf_a, S_sh.at[dst_v.at[j]], sem_sa).wait()

                @pl.when(jj < pn // 2 - 1)
                def _():
                    pltpu.async_copy(xl_hbm.at[src_v.at[j + 2]], buf_a, sem_ga)

                pltpu.make_async_copy(xl_hbm.at[src_v.at[j + 1]], buf_b, sem_gb).wait()
                pltpu.async_copy(buf_b, S_sh.at[dst_v.at[j + 1]], sem_sb, add=True)
                return carry

            lax.fori_loop(0, pn // 2, body, 0)
            pltpu.make_async_copy(buf_b, S_sh.at[dst_v.at[pn - 1]], sem_sb).wait()

        for c in range(NC):
            ch = SEG_CH[c]
            core_base = 0 if c == 0 else NS * SEG_CH[0]

            @pl.when(cid == c)
            def _(ch=ch, core_base=core_base):
                tile0 = core_base + sid * ch
                off = 0
                xl_hbm = xl0_hbm if c == 0 else xl1_hbm
                for pn in _pieces(ch):
                    run_piece(xl_hbm, tile0 + off, pn)
                    off += pn

        plsc.subcore_barrier()
        for off, sz in _ROW_BLOCKS:
            pltpu.sync_copy(S_sh.at[pl.ds(base + off, sz)], buf_a.at[pl.ds(0, sz)])
            pltpu.sync_copy(buf_a.at[pl.ds(0, sz)], out_hbm.at[cid, pl.ds(base + off, sz)])

    return _seg


_seg128 = _make_seg_kernel(OUT_CH)


TC_R = 1024


def _tc1_body(x_ref, w_ref, degp_ref, xl_ref, xlb_ref, dis_ref):
    deg = degp_ref[0, :, 0:1] + degp_ref[1, :, 0:1] + 1.0
    dis = lax.rsqrt(deg)
    xl = jnp.dot(x_ref[...], w_ref[...], preferred_element_type=jnp.float32)
    xlp = jnp.concatenate(
        [xl * dis, jnp.zeros((TC_R, OUT_CH - MID_CH), jnp.float32)], axis=1)
    xl_ref[...] = xlp
    xlb_ref[...] = xlp
    dis_ref[...] = dis


def _tc1(x, W1, degp):
    grid = N_PAD // TC_R
    return pl.pallas_call(
        _tc1_body,
        grid=(grid,),
        in_specs=[
            pl.BlockSpec((TC_R, IN_CH), lambda i: (i, 0)),
            pl.BlockSpec((IN_CH, MID_CH), lambda i: (0, 0)),
            pl.BlockSpec((NC, TC_R, DEG_W), lambda i: (0, i, 0)),
        ],
        out_specs=[
            pl.BlockSpec((TC_R, OUT_CH), lambda i: (i, 0)),
            pl.BlockSpec((TC_R, OUT_CH), lambda i: (i, 0)),
            pl.BlockSpec((TC_R, 1), lambda i: (i, 0)),
        ],
        out_shape=[
            jax.ShapeDtypeStruct((N_PAD, OUT_CH), jnp.float32),
            jax.ShapeDtypeStruct((N_PAD, OUT_CH), jnp.float32),
            jax.ShapeDtypeStruct((N_PAD, 1), jnp.float32),
        ],
    )(x, W1, degp)


def _tc2_body(s_ref, xl_ref, dis_ref, b_ref, w_ref, out_ref, outb_ref):
    dis = dis_ref[...]
    t = (s_ref[0] + s_ref[1] + xl_ref[...])[:, :MID_CH]
    z = dis * t + b_ref[...]
    h = jnp.where(z > 0, z, jnp.exp(z) - 1.0)
    o = jnp.dot(h, w_ref[...], preferred_element_type=jnp.float32) * dis
    out_ref[...] = o
    outb_ref[...] = o


def _tc2(S1, xl1, dis, b1, W2):
    grid = N_PAD // TC_R
    return pl.pallas_call(
        _tc2_body,
        grid=(grid,),
        in_specs=[
            pl.BlockSpec((NC, TC_R, OUT_CH), lambda i: (0, i, 0)),
            pl.BlockSpec((TC_R, OUT_CH), lambda i: (i, 0)),
            pl.BlockSpec((TC_R, 1), lambda i: (i, 0)),
            pl.BlockSpec((1, MID_CH), lambda i: (0, 0)),
            pl.BlockSpec((MID_CH, OUT_CH), lambda i: (0, 0)),
        ],
        out_specs=[pl.BlockSpec((TC_R, OUT_CH), lambda i: (i, 0)),
                   pl.BlockSpec((TC_R, OUT_CH), lambda i: (i, 0))],
        out_shape=[jax.ShapeDtypeStruct((N_PAD, OUT_CH), jnp.float32),
                   jax.ShapeDtypeStruct((N_PAD, OUT_CH), jnp.float32)],
    )(S1, xl1, dis, b1, W2)


def _tc3_body(s_ref, xl_ref, dis_ref, b_ref, out_ref):
    z = dis_ref[...] * (s_ref[0] + s_ref[1] + xl_ref[...]) + b_ref[...]
    m = jnp.max(z, axis=1, keepdims=True)
    e = jnp.exp(z - m)
    out_ref[...] = (z - m) - jnp.log(jnp.sum(e, axis=1, keepdims=True))


def _tc3(S2, xl2, dis, b2):
    grid = N_PAD // TC_R
    return pl.pallas_call(
        _tc3_body,
        grid=(grid,),
        in_specs=[
            pl.BlockSpec((NC, TC_R, OUT_CH), lambda i: (0, i, 0)),
            pl.BlockSpec((TC_R, OUT_CH), lambda i: (i, 0)),
            pl.BlockSpec((TC_R, 1), lambda i: (i, 0)),
            pl.BlockSpec((1, OUT_CH), lambda i: (0, 0)),
        ],
        out_specs=pl.BlockSpec((TC_R, OUT_CH), lambda i: (i, 0)),
        out_shape=jax.ShapeDtypeStruct((N_PAD, OUT_CH), jnp.float32),
    )(S2, xl2, dis, b2)


def kernel(node_feature, adj_list, W1, b1, W2, b2):
    src = adj_list[0].astype(jnp.int32)
    dst = adj_list[1].astype(jnp.int32)
    n_edges = src.shape[0]
    pad = E_PAD - n_edges
    src = jnp.concatenate([src, jnp.full((pad,), SCRAP, jnp.int32)])
    dst = jnp.concatenate([dst, jnp.full((pad,), SCRAP, jnp.int32)])
    srcs3 = src.reshape(NW, CHUNKS, CHUNK)
    dsts3 = dst.reshape(NW, CHUNKS, CHUNK)
    srcs2 = src.reshape(TOTAL_CHUNKS, CHUNK)
    dsts2 = dst.reshape(TOTAL_CHUNKS, CHUNK)

    x = jnp.zeros((N_PAD, IN_CH), jnp.float32).at[:N_NODES].set(node_feature)
    ones128 = jnp.ones((CHUNK, DEG_W), jnp.float32)
    zeros128 = jnp.zeros((CHUNK, OUT_CH), jnp.float32)

    degp = _deg_kernel(dsts3, ones128, zeros128)
    xl1, xl1b, dis = _tc1(x, W1, degp)
    S1 = _seg128(xl1, xl1b, srcs2, dsts2, zeros128)
    xl2, xl2b = _tc2(S1, xl1, dis, b1.reshape(1, MID_CH), W2)
    S2 = _seg128(xl2, xl2b, srcs2, dsts2, zeros128)
    out = _tc3(S2, xl2, dis, b2.reshape(1, OUT_CH))
    return out[:N_NODES]


# speedup vs baseline: 1.0589x; 1.0216x over previous
import functools

import jax
import jax.numpy as jnp
from jax import lax
from jax.experimental import pallas as pl
from jax.experimental.pallas import tpu as pltpu
from jax.experimental.pallas import tpu_sc as plsc

N_NODES = 10000
IN_CH = 128
MID_CH = 64
OUT_CH = 128

NC = 2
NS = 16
NW = NC * NS
CHUNK = 128
CHUNKS = 80
HALF_CHUNKS = 40
TOTAL_CHUNKS = NW * CHUNKS
SEG_CH = (152, 8)
E_PAD = NW * CHUNKS * CHUNK
N_PAD = 10240
N_SC = 10112
ROWS_PER_TILE = N_SC // NS
SCRAP = N_NODES

_mesh = plsc.VectorSubcoreMesh(core_axis_name="c", subcore_axis_name="s")

_ROW_BLOCKS = [(0, 128), (128, 128), (256, 128), (384, 128), (512, 120)]


DEG_W = 128


@functools.partial(
    pl.kernel,
    mesh=_mesh,
    out_type=jax.ShapeDtypeStruct((NC, N_PAD, DEG_W), jnp.float32),
    scratch_types=[
        pltpu.VMEM((CHUNKS, CHUNK), jnp.int32),
        pltpu.VMEM((CHUNK, DEG_W), jnp.float32),
        pltpu.VMEM((CHUNK, DEG_W), jnp.float32),
        pltpu.VMEM_SHARED((N_SC, DEG_W), jnp.float32),
        pltpu.SemaphoreType.DMA,
    ],
)
def _deg_kernel(dsts_hbm, ones_hbm, zeros_hbm, out_hbm, dst_v, ones_v, zbuf, S_sh,
                sem_s):
    cid = lax.axis_index("c")
    sid = lax.axis_index("s")
    wid = sid * NC + cid

    pltpu.sync_copy(zeros_hbm, zbuf)
    pltpu.sync_copy(ones_hbm, ones_v)
    pltpu.sync_copy(dsts_hbm.at[wid], dst_v)
    base = sid * ROWS_PER_TILE
    for off, sz in _ROW_BLOCKS:
        pltpu.sync_copy(zbuf.at[pl.ds(0, sz)], S_sh.at[pl.ds(base + off, sz)])
    plsc.subcore_barrier()

    def body(j, carry):
        pltpu.async_copy(ones_v, S_sh.at[dst_v.at[j]], sem_s, add=True)
        return carry

    lax.fori_loop(0, CHUNKS, body, 0)

    def drain(j, carry):
        pltpu.make_async_copy(ones_v, S_sh.at[dst_v.at[j]], sem_s).wait()
        return carry

    lax.fori_loop(0, CHUNKS, drain, 0)
    plsc.subcore_barrier()
    for off, sz in _ROW_BLOCKS:
        pltpu.sync_copy(S_sh.at[pl.ds(base + off, sz)], zbuf.at[pl.ds(0, sz)])
        pltpu.sync_copy(zbuf.at[pl.ds(0, sz)], out_hbm.at[cid, pl.ds(base + off, sz)])


def _pieces(n):
    return [min(HALF_CHUNKS, n - i) for i in range(0, n, HALF_CHUNKS)]


def _make_seg_kernel(C):
    @functools.partial(
        pl.kernel,
        mesh=_mesh,
        out_type=jax.ShapeDtypeStruct((NC, N_PAD, C), jnp.float32),
        scratch_types=[
            pltpu.VMEM((HALF_CHUNKS, CHUNK), jnp.int32),
            pltpu.VMEM((HALF_CHUNKS, CHUNK), jnp.int32),
            pltpu.VMEM((CHUNK, C), jnp.float32),
            pltpu.VMEM((CHUNK, C), jnp.float32),
            pltpu.VMEM_SHARED((N_SC, C), jnp.float32),
            pltpu.SemaphoreType.DMA,
            pltpu.SemaphoreType.DMA,
            pltpu.SemaphoreType.DMA,
            pltpu.SemaphoreType.DMA,
        ],
    )
    def _seg(xl0_hbm, xl1_hbm, srcs_hbm, dsts_hbm, zeros_hbm, out_hbm,
             src_v, dst_v, buf_a, buf_b, S_sh, sem_ga, sem_gb, sem_sa, sem_sb):
        cid = lax.axis_index("c")
        sid = lax.axis_index("s")
        base = sid * ROWS_PER_TILE

        pltpu.sync_copy(zeros_hbm, buf_a)
        for off, sz in _ROW_BLOCKS:
            pltpu.async_copy(buf_a.at[pl.ds(0, sz)], S_sh.at[pl.ds(base + off, sz)],
                             sem_sa)
        for off, sz in _ROW_BLOCKS:
            pltpu.make_async_copy(buf_a.at[pl.ds(0, sz)],
                                  S_sh.at[pl.ds(base + off, sz)], sem_sa).wait()
        plsc.subcore_barrier()

        def run_piece(xl_hbm, chunk0, pn):
            pltpu.sync_copy(srcs_hbm.at[pl.ds(chunk0, pn)], src_v.at[pl.ds(0, pn)])
            pltpu.sync_copy(dsts_hbm.at[pl.ds(chunk0, pn)], dst_v.at[pl.ds(0, pn)])
            pltpu.async_copy(xl_hbm.at[src_v.at[0]], buf_a, sem_ga)

            def body(jj, carry):
                j = 2 * jj

                @pl.when(jj > 0)
                def _():
                    pltpu.make_async_copy(buf_b, S_sh.at[dst_v.at[j - 1]],
                                          sem_sb).wait()

                pltpu.async_copy(xl_hbm.at[src_v.at[j + 1]], buf_b, sem_gb)
                pltpu.make_async_copy(xl_hbm.at[src_v.at[j]], buf_a, sem_ga).wait()
                pltpu.async_copy(buf_a, S_sh.at[dst_v.at[j]], sem_sa, add=True)
                pltpu.make_async_copy(buf_a, S_sh.at[dst_v.at[j]], sem_sa).wait()

                @pl.when(jj < pn // 2 - 1)
                def _():
                    pltpu.async_copy(xl_hbm.at[src_v.at[j + 2]], buf_a, sem_ga)

                pltpu.make_async_copy(xl_hbm.at[src_v.at[j + 1]], buf_b, sem_gb).wait()
                pltpu.async_copy(buf_b, S_sh.at[dst_v.at[j + 1]], sem_sb, add=True)
                return carry

            lax.fori_loop(0, pn // 2, body, 0)
            pltpu.make_async_copy(buf_b, S_sh.at[dst_v.at[pn - 1]], sem_sb).wait()

        for c in range(NC):
            ch = SEG_CH[c]
            core_base = 0 if c == 0 else NS * SEG_CH[0]

            @pl.when(cid == c)
            def _(ch=ch, core_base=core_base):
                tile0 = core_base + sid * ch
                off = 0
                xl_hbm = xl0_hbm if c == 0 else xl1_hbm
                for pn in _pieces(ch):
                    run_piece(xl_hbm, tile0 + off, pn)
                    off += pn

        plsc.subcore_barrier()
        bufs = [buf_a, buf_b]
        sems = [sem_ga, sem_gb]
        for k, (off, sz) in enumerate(_ROW_BLOCKS):
            b, sm = bufs[k % 2], sems[k % 2]
            if k >= 2:
                po, psz = _ROW_BLOCKS[k - 2]
                pltpu.make_async_copy(b.at[pl.ds(0, psz)],
                                      out_hbm.at[cid, pl.ds(base + po, psz)], sm).wait()
            pltpu.sync_copy(S_sh.at[pl.ds(base + off, sz)], b.at[pl.ds(0, sz)])
            pltpu.async_copy(b.at[pl.ds(0, sz)], out_hbm.at[cid, pl.ds(base + off, sz)], sm)
        for k in (len(_ROW_BLOCKS) - 2, len(_ROW_BLOCKS) - 1):
            off, sz = _ROW_BLOCKS[k]
            pltpu.make_async_copy(bufs[k % 2].at[pl.ds(0, sz)],
                                  out_hbm.at[cid, pl.ds(base + off, sz)],
                                  sems[k % 2]).wait()

    return _seg


_seg128 = _make_seg_kernel(OUT_CH)


TC_R = 1024


def _tc1_body(x_ref, w_ref, degp_ref, xl_ref, dis_ref):
    deg = degp_ref[0, :, 0:1] + degp_ref[1, :, 0:1] + 1.0
    dis = lax.rsqrt(deg)
    xl = jnp.dot(x_ref[...], w_ref[...], preferred_element_type=jnp.float32)
    xl_ref[...] = jnp.concatenate(
        [xl * dis, jnp.zeros((TC_R, OUT_CH - MID_CH), jnp.float32)], axis=1)
    dis_ref[...] = dis


def _tc1(x, W1, degp):
    grid = N_PAD // TC_R
    return pl.pallas_call(
        _tc1_body,
        grid=(grid,),
        in_specs=[
            pl.BlockSpec((TC_R, IN_CH), lambda i: (i, 0)),
            pl.BlockSpec((IN_CH, MID_CH), lambda i: (0, 0)),
            pl.BlockSpec((NC, TC_R, DEG_W), lambda i: (0, i, 0)),
        ],
        out_specs=[
            pl.BlockSpec((TC_R, OUT_CH), lambda i: (i, 0)),
            pl.BlockSpec((TC_R, 1), lambda i: (i, 0)),
        ],
        out_shape=[
            jax.ShapeDtypeStruct((N_PAD, OUT_CH), jnp.float32),
            jax.ShapeDtypeStruct((N_PAD, 1), jnp.float32),
        ],
    )(x, W1, degp)


def _tc2_body(s_ref, xl_ref, dis_ref, b_ref, w_ref, out_ref):
    dis = dis_ref[...]
    t = (s_ref[0] + s_ref[1] + xl_ref[...])[:, :MID_CH]
    z = dis * t + b_ref[...]
    h = jnp.where(z > 0, z, jnp.exp(z) - 1.0)
    out_ref[...] = jnp.dot(h, w_ref[...], preferred_element_type=jnp.float32) * dis


def _tc2(S1, xl1, dis, b1, W2):
    grid = N_PAD // TC_R
    return pl.pallas_call(
        _tc2_body,
        grid=(grid,),
        in_specs=[
            pl.BlockSpec((NC, TC_R, OUT_CH), lambda i: (0, i, 0)),
            pl.BlockSpec((TC_R, OUT_CH), lambda i: (i, 0)),
            pl.BlockSpec((TC_R, 1), lambda i: (i, 0)),
            pl.BlockSpec((1, MID_CH), lambda i: (0, 0)),
            pl.BlockSpec((MID_CH, OUT_CH), lambda i: (0, 0)),
        ],
        out_specs=pl.BlockSpec((TC_R, OUT_CH), lambda i: (i, 0)),
        out_shape=jax.ShapeDtypeStruct((N_PAD, OUT_CH), jnp.float32),
    )(S1, xl1, dis, b1, W2)


def _tc3_body(s_ref, xl_ref, dis_ref, b_ref, out_ref):
    z = dis_ref[...] * (s_ref[0] + s_ref[1] + xl_ref[...]) + b_ref[...]
    m = jnp.max(z, axis=1, keepdims=True)
    e = jnp.exp(z - m)
    out_ref[...] = (z - m) - jnp.log(jnp.sum(e, axis=1, keepdims=True))


def _tc3(S2, xl2, dis, b2):
    grid = N_PAD // TC_R
    return pl.pallas_call(
        _tc3_body,
        grid=(grid,),
        in_specs=[
            pl.BlockSpec((NC, TC_R, OUT_CH), lambda i: (0, i, 0)),
            pl.BlockSpec((TC_R, OUT_CH), lambda i: (i, 0)),
            pl.BlockSpec((TC_R, 1), lambda i: (i, 0)),
            pl.BlockSpec((1, OUT_CH), lambda i: (0, 0)),
        ],
        out_specs=pl.BlockSpec((TC_R, OUT_CH), lambda i: (i, 0)),
        out_shape=jax.ShapeDtypeStruct((N_NODES, OUT_CH), jnp.float32),
    )(S2, xl2, dis, b2)


def kernel(node_feature, adj_list, W1, b1, W2, b2):
    src = adj_list[0].astype(jnp.int32)
    dst = adj_list[1].astype(jnp.int32)
    n_edges = src.shape[0]
    pad = E_PAD - n_edges
    src = jnp.concatenate([src, jnp.full((pad,), SCRAP, jnp.int32)])
    dst = jnp.concatenate([dst, jnp.full((pad,), SCRAP, jnp.int32)])
    srcs3 = src.reshape(NW, CHUNKS, CHUNK)
    dsts3 = dst.reshape(NW, CHUNKS, CHUNK)
    srcs2 = src.reshape(TOTAL_CHUNKS, CHUNK)
    dsts2 = dst.reshape(TOTAL_CHUNKS, CHUNK)

    x = jnp.zeros((N_PAD, IN_CH), jnp.float32).at[:N_NODES].set(node_feature)
    ones128 = jnp.ones((CHUNK, DEG_W), jnp.float32)
    zeros128 = jnp.zeros((CHUNK, OUT_CH), jnp.float32)

    degp = _deg_kernel(dsts3, ones128, zeros128)
    xl1, dis = _tc1(x, W1, degp)
    S1 = _seg128(xl1, xl1, srcs2, dsts2, zeros128)
    xl2 = _tc2(S1, xl1, dis, b1.reshape(1, MID_CH), W2)
    S2 = _seg128(xl2, xl2, srcs2, dsts2, zeros128)
    return _tc3(S2, xl2, dis, b2.reshape(1, OUT_CH))


# speedup vs baseline: 1.0715x; 1.0119x over previous
import functools

import jax
import jax.numpy as jnp
from jax import lax
from jax.experimental import pallas as pl
from jax.experimental.pallas import tpu as pltpu
from jax.experimental.pallas import tpu_sc as plsc

N_NODES = 10000
IN_CH = 128
MID_CH = 64
OUT_CH = 128

NC = 2
NS = 16
NW = NC * NS
CHUNK = 128
CHUNKS = 80
HALF_CHUNKS = 40
TOTAL_CHUNKS = NW * CHUNKS
SEG_CH = (152, 8)
E_PAD = NW * CHUNKS * CHUNK
N_PAD = 10240
N_SC = 10112
ROWS_PER_TILE = N_SC // NS
SCRAP = N_NODES

_mesh = plsc.VectorSubcoreMesh(core_axis_name="c", subcore_axis_name="s")

_ROW_BLOCKS = [(0, 128), (128, 128), (256, 128), (384, 128), (512, 120)]


DEG_W = 128


@functools.partial(
    pl.kernel,
    mesh=_mesh,
    out_type=jax.ShapeDtypeStruct((NC, N_PAD, DEG_W), jnp.float32),
    scratch_types=[
        pltpu.VMEM((CHUNKS, CHUNK), jnp.int32),
        pltpu.VMEM((CHUNK, DEG_W), jnp.float32),
        pltpu.VMEM((CHUNK, DEG_W), jnp.float32),
        pltpu.VMEM_SHARED((N_SC, DEG_W), jnp.float32),
        pltpu.SemaphoreType.DMA,
    ],
)
def _deg_kernel(dsts_hbm, ones_hbm, zeros_hbm, out_hbm, dst_v, ones_v, zbuf, S_sh,
                sem_s):
    cid = lax.axis_index("c")
    sid = lax.axis_index("s")
    wid = sid * NC + cid

    pltpu.sync_copy(zeros_hbm, zbuf)
    pltpu.sync_copy(ones_hbm, ones_v)
    pltpu.sync_copy(dsts_hbm.at[wid], dst_v)
    base = sid * ROWS_PER_TILE
    for off, sz in _ROW_BLOCKS:
        pltpu.sync_copy(zbuf.at[pl.ds(0, sz)], S_sh.at[pl.ds(base + off, sz)])
    plsc.subcore_barrier()

    def body(j, carry):
        pltpu.async_copy(ones_v, S_sh.at[dst_v.at[j]], sem_s, add=True)
        return carry

    lax.fori_loop(0, CHUNKS, body, 0)

    def drain(j, carry):
        pltpu.make_async_copy(ones_v, S_sh.at[dst_v.at[j]], sem_s).wait()
        return carry

    lax.fori_loop(0, CHUNKS, drain, 0)
    plsc.subcore_barrier()
    obufs = [zbuf, ones_v]
    for k, (off, sz) in enumerate(_ROW_BLOCKS):
        b = obufs[k % 2]
        if k >= 2:
            po, psz = _ROW_BLOCKS[k - 2]
            pltpu.make_async_copy(b.at[pl.ds(0, psz)],
                                  out_hbm.at[cid, pl.ds(base + po, psz)], sem_s).wait()
        pltpu.sync_copy(S_sh.at[pl.ds(base + off, sz)], b.at[pl.ds(0, sz)])
        pltpu.async_copy(b.at[pl.ds(0, sz)], out_hbm.at[cid, pl.ds(base + off, sz)], sem_s)
    for k in (len(_ROW_BLOCKS) - 2, len(_ROW_BLOCKS) - 1):
        off, sz = _ROW_BLOCKS[k]
        pltpu.make_async_copy(obufs[k % 2].at[pl.ds(0, sz)],
                              out_hbm.at[cid, pl.ds(base + off, sz)], sem_s).wait()


def _pieces(n):
    return [min(HALF_CHUNKS, n - i) for i in range(0, n, HALF_CHUNKS)]


def _make_seg_kernel(C):
    @functools.partial(
        pl.kernel,
        mesh=_mesh,
        out_type=jax.ShapeDtypeStruct((NC, N_PAD, C), jnp.float32),
        scratch_types=[
            pltpu.VMEM((HALF_CHUNKS, CHUNK), jnp.int32),
            pltpu.VMEM((HALF_CHUNKS, CHUNK), jnp.int32),
            pltpu.VMEM((CHUNK, C), jnp.float32),
            pltpu.VMEM((CHUNK, C), jnp.float32),
            pltpu.VMEM_SHARED((N_SC, C), jnp.float32),
            pltpu.SemaphoreType.DMA,
            pltpu.SemaphoreType.DMA,
            pltpu.SemaphoreType.DMA,
            pltpu.SemaphoreType.DMA,
        ],
    )
    def _seg(xl0_hbm, xl1_hbm, srcs_hbm, dsts_hbm, zeros_hbm, out_hbm,
             src_v, dst_v, buf_a, buf_b, S_sh, sem_ga, sem_gb, sem_sa, sem_sb):
        cid = lax.axis_index("c")
        sid = lax.axis_index("s")
        base = sid * ROWS_PER_TILE

        pltpu.sync_copy(zeros_hbm, buf_a)
        for off, sz in _ROW_BLOCKS:
            pltpu.async_copy(buf_a.at[pl.ds(0, sz)], S_sh.at[pl.ds(base + off, sz)],
                             sem_sa)
        for off, sz in _ROW_BLOCKS:
            pltpu.make_async_copy(buf_a.at[pl.ds(0, sz)],
                                  S_sh.at[pl.ds(base + off, sz)], sem_sa).wait()
        plsc.subcore_barrier()

        def run_piece(xl_hbm, chunk0, pn):
            pltpu.sync_copy(srcs_hbm.at[pl.ds(chunk0, pn)], src_v.at[pl.ds(0, pn)])
            pltpu.sync_copy(dsts_hbm.at[pl.ds(chunk0, pn)], dst_v.at[pl.ds(0, pn)])
            pltpu.async_copy(xl_hbm.at[src_v.at[0]], buf_a, sem_ga)

            def body(jj, carry):
                j = 2 * jj

                @pl.when(jj > 0)
                def _():
                    pltpu.make_async_copy(buf_b, S_sh.at[dst_v.at[j - 1]],
                                          sem_sb).wait()

                pltpu.async_copy(xl_hbm.at[src_v.at[j + 1]], buf_b, sem_gb)
                pltpu.make_async_copy(xl_hbm.at[src_v.at[j]], buf_a, sem_ga).wait()
                pltpu.async_copy(buf_a, S_sh.at[dst_v.at[j]], sem_sa, add=True)
                pltpu.make_async_copy(buf_a, S_sh.at[dst_v.at[j]], sem_sa).wait()

                @pl.when(jj < pn // 2 - 1)
                def _():
                    pltpu.async_copy(xl_hbm.at[src_v.at[j + 2]], buf_a, sem_ga)

                pltpu.make_async_copy(xl_hbm.at[src_v.at[j + 1]], buf_b, sem_gb).wait()
                pltpu.async_copy(buf_b, S_sh.at[dst_v.at[j + 1]], sem_sb, add=True)
                return carry

            lax.fori_loop(0, pn // 2, body, 0)
            pltpu.make_async_copy(buf_b, S_sh.at[dst_v.at[pn - 1]], sem_sb).wait()

        for c in range(NC):
            ch = SEG_CH[c]
            core_base = 0 if c == 0 else NS * SEG_CH[0]

            @pl.when(cid == c)
            def _(ch=ch, core_base=core_base):
                tile0 = core_base + sid * ch
                off = 0
                xl_hbm = xl0_hbm if c == 0 else xl1_hbm
                for pn in _pieces(ch):
                    run_piece(xl_hbm, tile0 + off, pn)
                    off += pn

        plsc.subcore_barrier()
        bufs = [buf_a, buf_b]
        sems = [sem_ga, sem_gb]
        for k, (off, sz) in enumerate(_ROW_BLOCKS):
            b, sm = bufs[k % 2], sems[k % 2]
            if k >= 2:
                po, psz = _ROW_BLOCKS[k - 2]
                pltpu.make_async_copy(b.at[pl.ds(0, psz)],
                                      out_hbm.at[cid, pl.ds(base + po, psz)], sm).wait()
            pltpu.sync_copy(S_sh.at[pl.ds(base + off, sz)], b.at[pl.ds(0, sz)])
            pltpu.async_copy(b.at[pl.ds(0, sz)], out_hbm.at[cid, pl.ds(base + off, sz)], sm)
        for k in (len(_ROW_BLOCKS) - 2, len(_ROW_BLOCKS) - 1):
            off, sz = _ROW_BLOCKS[k]
            pltpu.make_async_copy(bufs[k % 2].at[pl.ds(0, sz)],
                                  out_hbm.at[cid, pl.ds(base + off, sz)],
                                  sems[k % 2]).wait()

    return _seg


_seg128 = _make_seg_kernel(OUT_CH)


TC_R = 1024


def _tc0_body(x_ref, w_ref, xlr_ref):
    xlr_ref[...] = jnp.dot(x_ref[...], w_ref[...],
                           preferred_element_type=jnp.float32)


def _tc0(x, W1):
    grid = N_PAD // TC_R
    return pl.pallas_call(
        _tc0_body,
        grid=(grid,),
        in_specs=[
            pl.BlockSpec((TC_R, IN_CH), lambda i: (i, 0)),
            pl.BlockSpec((IN_CH, MID_CH), lambda i: (0, 0)),
        ],
        out_specs=pl.BlockSpec((TC_R, MID_CH), lambda i: (i, 0)),
        out_shape=jax.ShapeDtypeStruct((N_PAD, MID_CH), jnp.float32),
    )(x, W1)


def _tc1_body(xlr_ref, degp_ref, xl_ref, dis_ref):
    deg = degp_ref[0, :, 0:1] + degp_ref[1, :, 0:1] + 1.0
    dis = lax.rsqrt(deg)
    xl_ref[...] = jnp.concatenate(
        [xlr_ref[...] * dis, jnp.zeros((TC_R, OUT_CH - MID_CH), jnp.float32)],
        axis=1)
    dis_ref[...] = dis


def _tc1(xlr, degp):
    grid = N_PAD // TC_R
    return pl.pallas_call(
        _tc1_body,
        grid=(grid,),
        in_specs=[
            pl.BlockSpec((TC_R, MID_CH), lambda i: (i, 0)),
            pl.BlockSpec((NC, TC_R, DEG_W), lambda i: (0, i, 0)),
        ],
        out_specs=[
            pl.BlockSpec((TC_R, OUT_CH), lambda i: (i, 0)),
            pl.BlockSpec((TC_R, 1), lambda i: (i, 0)),
        ],
        out_shape=[
            jax.ShapeDtypeStruct((N_PAD, OUT_CH), jnp.float32),
            jax.ShapeDtypeStruct((N_PAD, 1), jnp.float32),
        ],
    )(xlr, degp)


def _tc2_body(s_ref, xl_ref, dis_ref, b_ref, w_ref, out_ref):
    dis = dis_ref[...]
    t = (s_ref[0] + s_ref[1] + xl_ref[...])[:, :MID_CH]
    z = dis * t + b_ref[...]
    h = jnp.where(z > 0, z, jnp.exp(z) - 1.0)
    out_ref[...] = jnp.dot(h, w_ref[...], preferred_element_type=jnp.float32) * dis


def _tc2(S1, xl1, dis, b1, W2):
    grid = N_PAD // TC_R
    return pl.pallas_call(
        _tc2_body,
        grid=(grid,),
        in_specs=[
            pl.BlockSpec((NC, TC_R, OUT_CH), lambda i: (0, i, 0)),
            pl.BlockSpec((TC_R, OUT_CH), lambda i: (i, 0)),
            pl.BlockSpec((TC_R, 1), lambda i: (i, 0)),
            pl.BlockSpec((1, MID_CH), lambda i: (0, 0)),
            pl.BlockSpec((MID_CH, OUT_CH), lambda i: (0, 0)),
        ],
        out_specs=pl.BlockSpec((TC_R, OUT_CH), lambda i: (i, 0)),
        out_shape=jax.ShapeDtypeStruct((N_PAD, OUT_CH), jnp.float32),
    )(S1, xl1, dis, b1, W2)


def _tc3_body(s_ref, xl_ref, dis_ref, b_ref, out_ref):
    z = dis_ref[...] * (s_ref[0] + s_ref[1] + xl_ref[...]) + b_ref[...]
    m = jnp.max(z, axis=1, keepdims=True)
    e = jnp.exp(z - m)
    out_ref[...] = (z - m) - jnp.log(jnp.sum(e, axis=1, keepdims=True))


def _tc3(S2, xl2, dis, b2):
    grid = N_PAD // TC_R
    return pl.pallas_call(
        _tc3_body,
        grid=(grid,),
        in_specs=[
            pl.BlockSpec((NC, TC_R, OUT_CH), lambda i: (0, i, 0)),
            pl.BlockSpec((TC_R, OUT_CH), lambda i: (i, 0)),
            pl.BlockSpec((TC_R, 1), lambda i: (i, 0)),
            pl.BlockSpec((1, OUT_CH), lambda i: (0, 0)),
        ],
        out_specs=pl.BlockSpec((TC_R, OUT_CH), lambda i: (i, 0)),
        out_shape=jax.ShapeDtypeStruct((N_NODES, OUT_CH), jnp.float32),
    )(S2, xl2, dis, b2)


def kernel(node_feature, adj_list, W1, b1, W2, b2):
    src = adj_list[0].astype(jnp.int32)
    dst = adj_list[1].astype(jnp.int32)
    n_edges = src.shape[0]
    pad = E_PAD - n_edges
    src = jnp.concatenate([src, jnp.full((pad,), SCRAP, jnp.int32)])
    dst = jnp.concatenate([dst, jnp.full((pad,), SCRAP, jnp.int32)])
    srcs3 = src.reshape(NW, CHUNKS, CHUNK)
    dsts3 = dst.reshape(NW, CHUNKS, CHUNK)
    srcs2 = src.reshape(TOTAL_CHUNKS, CHUNK)
    dsts2 = dst.reshape(TOTAL_CHUNKS, CHUNK)

    x = jnp.zeros((N_PAD, IN_CH), jnp.float32).at[:N_NODES].set(node_feature)
    ones128 = jnp.ones((CHUNK, DEG_W), jnp.float32)
    zeros128 = jnp.zeros((CHUNK, OUT_CH), jnp.float32)

    xlr = _tc0(x, W1)
    degp = _deg_kernel(dsts3, ones128, zeros128)
    xl1, dis = _tc1(xlr, degp)
    S1 = _seg128(xl1, xl1, srcs2, dsts2, zeros128)
    xl2 = _tc2(S1, xl1, dis, b1.reshape(1, MID_CH), W2)
    S2 = _seg128(xl2, xl2, srcs2, dsts2, zeros128)
    return _tc3(S2, xl2, dis, b2.reshape(1, OUT_CH))
